# baseline (speedup 1.0000x reference)
import jax
import jax.numpy as jnp
from jax.experimental import pallas as pl

_NB = 512


def _block_kernel(xyz_ref, pt_ref, feats_ref, out_ref):
    x = xyz_ref[...]
    p = pt_ref[...]
    m = x.shape[0]
    nb = p.shape[1]

    xp = jnp.dot(
        x, p,
        preferred_element_type=jnp.float32,
        precision=jax.lax.Precision.HIGHEST,
    )
    xn = jnp.sum(x * x, axis=1, keepdims=True)
    pn = jnp.sum(p * p, axis=0, keepdims=True)
    d2 = jnp.maximum((xn + pn) - 2.0 * xp, 0.0)

    h = m // 2
    a, b = d2[:h], d2[h:]
    s1 = jnp.minimum(a, b)
    s2 = jnp.maximum(a, b)
    q = h // 2
    a1, a2 = s1[:q], s2[:q]
    b1, b2 = s1[q:], s2[q:]
    k1 = jnp.minimum(a1, b1)
    v = jnp.maximum(a1, b1)
    u = jnp.minimum(a2, b2)
    k2 = jnp.minimum(v, u)
    k3 = jnp.maximum(v, u)
    r = q // 2
    while r >= 1:
        a1, a2, a3 = k1[:r], k2[:r], k3[:r]
        b1, b2, b3 = k1[r:], k2[r:], k3[r:]
        n1 = jnp.minimum(a1, b1)
        n2 = jnp.minimum(jnp.minimum(a2, b2), jnp.maximum(a1, b1))
        n3 = jnp.minimum(
            jnp.minimum(a3, b3),
            jnp.minimum(jnp.maximum(a2, b1), jnp.maximum(a1, b2)),
        )
        k1, k2, k3 = n1, n2, n3
        r //= 2

    inv1 = 1.0 / (k1 + 1e-8)
    inv2 = 1.0 / (k2 + 1e-8)
    inv3 = 1.0 / (k3 + 1e-8)
    norm = inv1 + inv2 + inv3

    wt = (
        jnp.where(d2 == k1, inv1 / norm, 0.0)
        + jnp.where(d2 == k2, inv2 / norm, 0.0)
        + jnp.where(d2 == k3, inv3 / norm, 0.0)
    )
    out_ref[...] = jnp.dot(
        feats_ref[...], wt, preferred_element_type=jnp.float32
    )


@jax.jit
def kernel(xyz, parent_xyz, feats):
    bs, m, _ = xyz.shape
    n = parent_xyz.shape[1]
    c = feats.shape[1]
    parent_t = jnp.transpose(parent_xyz, (0, 2, 1))
    grid = (bs, n // _NB)
    return pl.pallas_call(
        _block_kernel,
        grid=grid,
        in_specs=[
            pl.BlockSpec((None, m, 3), lambda b, i: (b, 0, 0)),
            pl.BlockSpec((None, 3, _NB), lambda b, i: (b, 0, i)),
            pl.BlockSpec((None, c, m), lambda b, i: (b, 0, 0)),
        ],
        out_specs=pl.BlockSpec((None, c, _NB), lambda b, i: (b, 0, i)),
        out_shape=jax.ShapeDtypeStruct((bs, c, n), jnp.float32),
    )(xyz, parent_t, feats)


# speedup vs baseline: 1.2907x; 1.2907x over previous
import jax
import jax.numpy as jnp
from jax.experimental import pallas as pl

_NB = 512


def _block_kernel(xyz_ref, pt_ref, feats_ref, out_ref):
    x = xyz_ref[...]
    p = pt_ref[...]
    m = x.shape[0]
    nb = p.shape[1]

    xp = (
        x[:, 0:1] * p[0:1, :]
        + x[:, 1:2] * p[1:2, :]
        + x[:, 2:3] * p[2:3, :]
    )
    xn = jnp.sum(x * x, axis=1, keepdims=True)
    pn = jnp.sum(p * p, axis=0, keepdims=True)
    d2 = jnp.maximum((xn + pn) - 2.0 * xp, 0.0)

    h = m // 2
    a, b = d2[:h], d2[h:]
    s1 = jnp.minimum(a, b)
    s2 = jnp.maximum(a, b)
    q = h // 2
    a1, a2 = s1[:q], s2[:q]
    b1, b2 = s1[q:], s2[q:]
    k1 = jnp.minimum(a1, b1)
    v = jnp.maximum(a1, b1)
    u = jnp.minimum(a2, b2)
    k2 = jnp.minimum(v, u)
    k3 = jnp.maximum(v, u)
    r = q // 2
    while r >= 1:
        a1, a2, a3 = k1[:r], k2[:r], k3[:r]
        b1, b2, b3 = k1[r:], k2[r:], k3[r:]
        n1 = jnp.minimum(a1, b1)
        n2 = jnp.minimum(jnp.minimum(a2, b2), jnp.maximum(a1, b1))
        n3 = jnp.minimum(
            jnp.minimum(a3, b3),
            jnp.minimum(jnp.maximum(a2, b1), jnp.maximum(a1, b2)),
        )
        k1, k2, k3 = n1, n2, n3
        r //= 2

    inv1 = 1.0 / (k1 + 1e-8)
    inv2 = 1.0 / (k2 + 1e-8)
    inv3 = 1.0 / (k3 + 1e-8)
    norm = inv1 + inv2 + inv3

    wt = (
        jnp.where(d2 == k1, inv1 / norm, 0.0)
        + jnp.where(d2 == k2, inv2 / norm, 0.0)
        + jnp.where(d2 == k3, inv3 / norm, 0.0)
    )
    out_ref[...] = jnp.dot(
        feats_ref[...], wt, preferred_element_type=jnp.float32
    )


@jax.jit
def kernel(xyz, parent_xyz, feats):
    bs, m, _ = xyz.shape
    n = parent_xyz.shape[1]
    c = feats.shape[1]
    parent_t = jnp.transpose(parent_xyz, (0, 2, 1))
    grid = (bs, n // _NB)
    return pl.pallas_call(
        _block_kernel,
        grid=grid,
        in_specs=[
            pl.BlockSpec((None, m, 3), lambda b, i: (b, 0, 0)),
            pl.BlockSpec((None, 3, _NB), lambda b, i: (b, 0, i)),
            pl.BlockSpec((None, c, m), lambda b, i: (b, 0, 0)),
        ],
        out_specs=pl.BlockSpec((None, c, _NB), lambda b, i: (b, 0, i)),
        out_shape=jax.ShapeDtypeStruct((bs, c, n), jnp.float32),
    )(xyz, parent_t, feats)


# speedup vs baseline: 1.4037x; 1.0875x over previous
import jax
import jax.numpy as jnp
from jax.experimental import pallas as pl

_NB = 512


def _block_kernel(xyz_ref, pt_ref, feats_ref, out_ref):
    x = xyz_ref[...]
    p = pt_ref[...]
    m = x.shape[0]
    nb = p.shape[1]

    xp = (
        x[:, 0:1] * p[0:1, :]
        + x[:, 1:2] * p[1:2, :]
        + x[:, 2:3] * p[2:3, :]
    )
    xn = jnp.sum(x * x, axis=1, keepdims=True)
    pn = jnp.sum(p * p, axis=0, keepdims=True)
    d2 = jnp.maximum((xn + pn) - 2.0 * xp, 0.0)

    h = m // 2
    a, b = d2[:h], d2[h:]
    s1 = jnp.minimum(a, b)
    s2 = jnp.maximum(a, b)
    q = h // 2
    a1, a2 = s1[:q], s2[:q]
    b1, b2 = s1[q:], s2[q:]
    k1 = jnp.minimum(a1, b1)
    v = jnp.maximum(a1, b1)
    u = jnp.minimum(a2, b2)
    k2 = jnp.minimum(v, u)
    k3 = jnp.maximum(v, u)
    r = q // 2
    while r >= 1:
        a1, a2, a3 = k1[:r], k2[:r], k3[:r]
        b1, b2, b3 = k1[r:], k2[r:], k3[r:]
        n1 = jnp.minimum(a1, b1)
        n2 = jnp.minimum(jnp.minimum(a2, b2), jnp.maximum(a1, b1))
        n3 = jnp.minimum(
            jnp.minimum(a3, b3),
            jnp.minimum(jnp.maximum(a2, b1), jnp.maximum(a1, b2)),
        )
        k1, k2, k3 = n1, n2, n3
        r //= 2

    inv1 = 1.0 / (k1 + 1e-8)
    inv2 = 1.0 / (k2 + 1e-8)
    inv3 = 1.0 / (k3 + 1e-8)
    invnorm = 1.0 / (inv1 + inv2 + inv3)

    wt = jnp.where(d2 <= k3, invnorm / (d2 + 1e-8), 0.0)
    out_ref[...] = jnp.dot(
        feats_ref[...], wt, preferred_element_type=jnp.float32
    )


@jax.jit
def kernel(xyz, parent_xyz, feats):
    bs, m, _ = xyz.shape
    n = parent_xyz.shape[1]
    c = feats.shape[1]
    parent_t = jnp.transpose(parent_xyz, (0, 2, 1))
    grid = (bs, n // _NB)
    return pl.pallas_call(
        _block_kernel,
        grid=grid,
        in_specs=[
            pl.BlockSpec((None, m, 3), lambda b, i: (b, 0, 0)),
            pl.BlockSpec((None, 3, _NB), lambda b, i: (b, 0, i)),
            pl.BlockSpec((None, c, m), lambda b, i: (b, 0, 0)),
        ],
        out_specs=pl.BlockSpec((None, c, _NB), lambda b, i: (b, 0, i)),
        out_shape=jax.ShapeDtypeStruct((bs, c, n), jnp.float32),
    )(xyz, parent_t, feats)


# speedup vs baseline: 1.5028x; 1.0706x over previous
import jax
import jax.numpy as jnp
from jax.experimental import pallas as pl

_NB = 512
_K = 24


def _block_kernel(a_ref, b_ref, feats_ref, out_ref):
    a = a_ref[...]
    b = b_ref[...]
    m = a.shape[0]

    d2 = jnp.maximum(
        jnp.dot(a, b, preferred_element_type=jnp.float32), 0.0
    )

    h = m // 2
    s1 = jnp.minimum(d2[:h], d2[h:])
    s2 = jnp.maximum(d2[:h], d2[h:])
    q = h // 2
    a1, a2 = s1[:q], s2[:q]
    b1, b2 = s1[q:], s2[q:]
    k1 = jnp.minimum(a1, b1)
    v = jnp.maximum(a1, b1)
    u = jnp.minimum(a2, b2)
    k2 = jnp.minimum(v, u)
    k3 = jnp.maximum(v, u)
    r = q // 2
    while r >= 1:
        a1, a2, a3 = k1[:r], k2[:r], k3[:r]
        b1, b2, b3 = k1[r:], k2[r:], k3[r:]
        n1 = jnp.minimum(a1, b1)
        n2 = jnp.minimum(jnp.minimum(a2, b2), jnp.maximum(a1, b1))
        n3 = jnp.minimum(
            jnp.minimum(a3, b3),
            jnp.minimum(jnp.maximum(a2, b1), jnp.maximum(a1, b2)),
        )
        k1, k2, k3 = n1, n2, n3
        r //= 2

    inv1 = 1.0 / (k1 + 1e-8)
    inv2 = 1.0 / (k2 + 1e-8)
    inv3 = 1.0 / (k3 + 1e-8)
    invnorm = 1.0 / (inv1 + inv2 + inv3)

    wt = jnp.where(d2 <= k3, invnorm / (d2 + 1e-8), 0.0)
    out_ref[...] = jnp.dot(
        feats_ref[...], wt, preferred_element_type=jnp.float32
    )


def _hi_lo(x):
    hi = x.astype(jnp.bfloat16)
    rem = x - hi.astype(jnp.float32)
    lo = rem.astype(jnp.bfloat16)
    lo2 = (rem - lo.astype(jnp.float32)).astype(jnp.bfloat16)
    return hi, lo, lo2


@jax.jit
def kernel(xyz, parent_xyz, feats):
    bs, m, _ = xyz.shape
    n = parent_xyz.shape[1]
    c = feats.shape[1]

    xh, xl, xl2 = _hi_lo(xyz)
    xn = jnp.sum(xyz * xyz, axis=2)
    xn1, xn2, xn3 = _hi_lo(xn)
    ones_x = jnp.ones((bs, m, 3), jnp.bfloat16)
    a_cat = jnp.concatenate(
        [
            -2.0 * xh, -2.0 * xh, -2.0 * xl, -2.0 * xl, -2.0 * xh,
            -2.0 * xl2,
            xn1[..., None], xn2[..., None], xn3[..., None],
            ones_x,
        ],
        axis=2,
    )

    parent_t = jnp.transpose(parent_xyz, (0, 2, 1))
    ph, plo, plo2 = _hi_lo(parent_t)
    pn = jnp.sum(parent_t * parent_t, axis=1)
    pn1, pn2, pn3 = _hi_lo(pn)
    ones_p = jnp.ones((bs, 3, n), jnp.bfloat16)
    b_cat = jnp.concatenate(
        [
            ph, plo, ph, plo, plo2, ph,
            ones_p,
            pn1[:, None, :], pn2[:, None, :], pn3[:, None, :],
        ],
        axis=1,
    )

    grid = (bs, n // _NB)
    return pl.pallas_call(
        _block_kernel,
        grid=grid,
        in_specs=[
            pl.BlockSpec((None, m, _K), lambda b, i: (b, 0, 0)),
            pl.BlockSpec((None, _K, _NB), lambda b, i: (b, 0, i)),
            pl.BlockSpec((None, c, m), lambda b, i: (b, 0, 0)),
        ],
        out_specs=pl.BlockSpec((None, c, _NB), lambda b, i: (b, 0, i)),
        out_shape=jax.ShapeDtypeStruct((bs, c, n), jnp.float32),
    )(a_cat, b_cat, feats)


# speedup vs baseline: 1.7430x; 1.1598x over previous
import jax
import jax.numpy as jnp
from jax.experimental import pallas as pl

_NB = 1024


def _block_kernel(xyz_ref, pt_ref, feats_ref, out_ref):
    x = xyz_ref[...]
    p = pt_ref[...]
    m = x.shape[0]

    xp = (
        x[:, 0:1] * p[0:1, :]
        + x[:, 1:2] * p[1:2, :]
        + x[:, 2:3] * p[2:3, :]
    )
    xn = jnp.sum(x * x, axis=1, keepdims=True)
    pn = jnp.sum(p * p, axis=0, keepdims=True)
    d2 = jnp.maximum((xn + pn) - 2.0 * xp, 0.0)

    h = m // 2
    s1 = jnp.minimum(d2[:h], d2[h:])
    s2 = jnp.maximum(d2[:h], d2[h:])
    q = h // 2
    a1, a2 = s1[:q], s2[:q]
    b1, b2 = s1[q:], s2[q:]
    k1 = jnp.minimum(a1, b1)
    v = jnp.maximum(a1, b1)
    u = jnp.minimum(a2, b2)
    k2 = jnp.minimum(v, u)
    k3 = jnp.maximum(v, u)
    r = q // 2
    while r >= 1:
        a1, a2, a3 = k1[:r], k2[:r], k3[:r]
        b1, b2, b3 = k1[r:], k2[r:], k3[r:]
        n1 = jnp.minimum(a1, b1)
        n2 = jnp.minimum(jnp.minimum(a2, b2), jnp.maximum(a1, b1))
        n3 = jnp.minimum(
            jnp.minimum(a3, b3),
            jnp.minimum(jnp.maximum(a2, b1), jnp.maximum(a1, b2)),
        )
        k1, k2, k3 = n1, n2, n3
        r //= 2

    inv1 = 1.0 / (k1 + 1e-8)
    inv2 = 1.0 / (k2 + 1e-8)
    inv3 = 1.0 / (k3 + 1e-8)
    invnorm = 1.0 / (inv1 + inv2 + inv3)

    wt = jnp.where(d2 <= k3, invnorm / (d2 + 1e-8), 0.0)
    out_ref[...] = jnp.dot(
        feats_ref[...], wt, preferred_element_type=jnp.float32
    )


@jax.jit
def kernel(xyz, parent_xyz, feats):
    bs, m, _ = xyz.shape
    n = parent_xyz.shape[1]
    c = feats.shape[1]
    parent_t = jnp.transpose(parent_xyz, (0, 2, 1))
    grid = (bs, n // _NB)
    return pl.pallas_call(
        _block_kernel,
        grid=grid,
        in_specs=[
            pl.BlockSpec((None, m, 3), lambda b, i: (b, 0, 0)),
            pl.BlockSpec((None, 3, _NB), lambda b, i: (b, 0, i)),
            pl.BlockSpec((None, c, m), lambda b, i: (b, 0, 0)),
        ],
        out_specs=pl.BlockSpec((None, c, _NB), lambda b, i: (b, 0, i)),
        out_shape=jax.ShapeDtypeStruct((bs, c, n), jnp.float32),
    )(xyz, parent_t, feats)
